# baseline (speedup 1.0000x reference)
import functools

import jax
import jax.numpy as jnp
from jax import lax
from jax.experimental import pallas as pl
from jax.experimental.pallas import tpu as pltpu
from jax.experimental.pallas import tpu_sc as plsc

_V = 49408
_D = 768
_T = 77
_B = 1024
_NW = 32
_BPW = _B // _NW
_TP = 80
_LANES = 16
_DV = _D // _LANES

_NS = 6
_T8 = (0, 16, 32, 48, 64, 72)
_GR = (16, 16, 16, 16, 8, 8)
_SR = (16, 16, 16, 16, 8, 5)
_NCHUNK = _BPW * _NS

_mesh = plsc.VectorSubcoreMesh(core_axis_name="c", subcore_axis_name="s")


@functools.partial(
    pl.kernel,
    out_type=jax.ShapeDtypeStruct((_B, _T, _D), jnp.float32),
    mesh=_mesh,
    scratch_types=(
        [pltpu.VMEM((_BPW * _TP,), jnp.int32),
         pltpu.VMEM((_T * _D,), jnp.float32)]
        + [pltpu.VMEM((r, _D), jnp.float32) for r in _GR]
        + [pltpu.SemaphoreType.DMA for _ in range(2 * _NS)]
    ),
)
def _embed_sc(tok_ref, pos_ref, tab_ref, out_ref, idx_v, pos_v, *rest):
    bufs = rest[:_NS]
    gsems = rest[_NS:2 * _NS]
    ssems = rest[2 * _NS:3 * _NS]

    wid = lax.axis_index("s") * 2 + lax.axis_index("c")
    b_base = wid * _BPW

    pltpu.sync_copy(tok_ref.at[pl.ds(b_base * _TP, _BPW * _TP)], idx_v)
    pltpu.sync_copy(pos_ref, pos_v)

    def fire_gather(be, s):
        pltpu.async_copy(
            tab_ref.at[idx_v.at[pl.ds(be * _TP + _T8[s], _GR[s])]],
            bufs[s], gsems[s])

    def wait_gather(be, s):
        pltpu.make_async_copy(
            tab_ref.at[idx_v.at[pl.ds(be * _TP + _T8[s], _GR[s])]],
            bufs[s], gsems[s]).wait()

    def add_pos(s):
        def dv_body(dv, carry):
            off = dv * _LANES
            for r in range(_SR[s]):
                pv = pos_v[pl.ds((_T8[s] + r) * _D + off, _LANES)]
                plsc.addupdate(bufs[s].at[r, pl.ds(off, _LANES)], pv)
            return carry
        lax.fori_loop(0, _DV, dv_body, 0)

    def fire_store(be, s):
        pltpu.async_copy(
            bufs[s].at[pl.ds(0, _SR[s])],
            out_ref.at[b_base + be, pl.ds(_T8[s], _SR[s])], ssems[s])

    def wait_store(be, s):
        pltpu.make_async_copy(
            bufs[s].at[pl.ds(0, _SR[s])],
            out_ref.at[b_base + be, pl.ds(_T8[s], _SR[s])], ssems[s]).wait()

    for s in range(_NS):
        fire_gather(0, s)

    def be_body(be, carry):
        for s in range(_NS):
            c = be * _NS + s
            s2 = (s - 2) % _NS
            ber = be - (1 if s < 2 else 0)
            bef = be + (1 if s >= 2 else 0)

            @pl.when(c >= 2)
            def _retire():
                wait_store(ber, s2)

            @pl.when(jnp.logical_and(c >= 2, c + 4 <= _NCHUNK - 1))
            def _refill():
                fire_gather(bef, s2)

            wait_gather(be, s)
            add_pos(s)
            fire_store(be, s)

        return carry

    lax.fori_loop(0, _BPW, be_body, 0)

    wait_store(_BPW - 1, 4)
    wait_store(_BPW - 1, 5)


def kernel(tokens, token_embd, position_embd):
    tokens_pad = jnp.pad(tokens.astype(jnp.int32), ((0, 0), (0, _TP - _T)))
    return _embed_sc(tokens_pad.reshape(-1), position_embd.reshape(-1),
                     token_embd)


# speedup vs baseline: 1.0338x; 1.0338x over previous
import functools

import jax
import jax.numpy as jnp
from jax import lax
from jax.experimental import pallas as pl
from jax.experimental.pallas import tpu as pltpu
from jax.experimental.pallas import tpu_sc as plsc

_V = 49408
_D = 768
_T = 77
_B = 1024
_NW = 32
_BPW = _B // _NW
_TP = 80
_LANES = 16
_DV = _D // _LANES

_NS = 6
_T8 = (0, 16, 32, 48, 64, 72)
_GR = (16, 16, 16, 16, 8, 8)
_SR = (16, 16, 16, 16, 8, 5)
_NCHUNK = _BPW * _NS

_mesh = plsc.VectorSubcoreMesh(core_axis_name="c", subcore_axis_name="s")


@functools.partial(
    pl.kernel,
    out_type=jax.ShapeDtypeStruct((_B, _T, _D), jnp.float32),
    mesh=_mesh,
    scratch_types=(
        [pltpu.VMEM((_BPW * _TP,), jnp.int32),
         pltpu.VMEM((_T * _D,), jnp.float32)]
        + [pltpu.VMEM((r, _D), jnp.float32) for r in _GR]
        + [pltpu.SemaphoreType.DMA for _ in range(2 * _NS)]
    ),
)
def _embed_sc(tok_ref, pos_ref, tab_ref, out_ref, idx_v, pos_v, *rest):
    bufs = rest[:_NS]
    gsems = rest[_NS:2 * _NS]
    ssems = rest[2 * _NS:3 * _NS]

    wid = lax.axis_index("s") * 2 + lax.axis_index("c")
    b_base = wid * _BPW

    pltpu.sync_copy(tok_ref.at[pl.ds(b_base * _TP, _BPW * _TP)], idx_v)
    pltpu.sync_copy(pos_ref, pos_v)

    def fire_gather(be, s):
        pltpu.async_copy(
            tab_ref.at[idx_v.at[pl.ds(be * _TP + _T8[s], _GR[s])]],
            bufs[s], gsems[s])

    def wait_gather(be, s):
        pltpu.make_async_copy(
            tab_ref.at[idx_v.at[pl.ds(be * _TP + _T8[s], _GR[s])]],
            bufs[s], gsems[s]).wait()

    def add_pos(s):
        nr = _SR[s]

        def pload(r, dv):
            return pos_v[pl.ds((_T8[s] + r) * _D + dv * _LANES, _LANES)]

        cur = tuple(pload(r, 0) for r in range(nr))

        def dv_body(dv, cur):
            nxt = []
            for r in range(nr):
                plsc.addupdate(bufs[s].at[r, pl.ds(dv * _LANES, _LANES)],
                               cur[r])
                nxt.append(pos_v[pl.ds(
                    (_T8[s] + r) * _D + (dv + 1) * _LANES, _LANES)])
            return tuple(nxt)

        cur = lax.fori_loop(0, _DV - 1, dv_body, cur)
        for r in range(nr):
            plsc.addupdate(
                bufs[s].at[r, pl.ds((_DV - 1) * _LANES, _LANES)], cur[r])

    def fire_store(be, s):
        pltpu.async_copy(
            bufs[s].at[pl.ds(0, _SR[s])],
            out_ref.at[b_base + be, pl.ds(_T8[s], _SR[s])], ssems[s])

    def wait_store(be, s):
        pltpu.make_async_copy(
            bufs[s].at[pl.ds(0, _SR[s])],
            out_ref.at[b_base + be, pl.ds(_T8[s], _SR[s])], ssems[s]).wait()

    for s in range(_NS):
        fire_gather(0, s)

    def be_body(be, carry):
        for s in range(_NS):
            c = be * _NS + s
            s2 = (s - 2) % _NS
            ber = be - (1 if s < 2 else 0)
            bef = be + (1 if s >= 2 else 0)

            @pl.when(c >= 2)
            def _retire():
                wait_store(ber, s2)

            @pl.when(jnp.logical_and(c >= 2, c + 4 <= _NCHUNK - 1))
            def _refill():
                fire_gather(bef, s2)

            wait_gather(be, s)
            add_pos(s)
            fire_store(be, s)

        return carry

    lax.fori_loop(0, _BPW, be_body, 0)

    wait_store(_BPW - 1, 4)
    wait_store(_BPW - 1, 5)


def kernel(tokens, token_embd, position_embd):
    tokens_pad = jnp.pad(tokens.astype(jnp.int32), ((0, 0), (0, _TP - _T)))
    return _embed_sc(tokens_pad.reshape(-1), position_embd.reshape(-1),
                     token_embd)


# speedup vs baseline: 1.0559x; 1.0213x over previous
import functools

import jax
import jax.numpy as jnp
from jax import lax
from jax.experimental import pallas as pl
from jax.experimental.pallas import tpu as pltpu
from jax.experimental.pallas import tpu_sc as plsc

_V = 49408
_D = 768
_T = 77
_B = 1024
_NW = 32
_BPW = _B // _NW
_NPAIR = _BPW // 2
_TP = 80
_LANES = 16
_DV = _D // _LANES

_NT = 10
_T8 = tuple(8 * i for i in range(_NT))
_SR = (8,) * 9 + (5,)
_NSLOT = 5
_NCHUNK = _NPAIR * _NT

_mesh = plsc.VectorSubcoreMesh(core_axis_name="c", subcore_axis_name="s")


@functools.partial(
    pl.kernel,
    out_type=jax.ShapeDtypeStruct((_B, _T, _D), jnp.float32),
    mesh=_mesh,
    scratch_types=(
        [pltpu.VMEM((_BPW * _TP,), jnp.int32),
         pltpu.VMEM((_T * _D,), jnp.float32)]
        + [pltpu.VMEM((16, _D), jnp.float32) for _ in range(_NSLOT)]
        + [pltpu.SemaphoreType.DMA for _ in range(2 * _NSLOT)]
    ),
)
def _embed_sc(tok_ref, pos_ref, tab_ref, out_ref, idx_v, pos_v, *rest):
    bufs = rest[:_NSLOT]
    gsems = rest[_NSLOT:2 * _NSLOT]
    ssems = rest[2 * _NSLOT:3 * _NSLOT]

    wid = lax.axis_index("s") * 2 + lax.axis_index("c")
    b_base = wid * _BPW

    pltpu.sync_copy(tok_ref.at[pl.ds(b_base * _TP, _BPW * _TP)], idx_v)
    pltpu.sync_copy(pos_ref, pos_v)

    def halves(p, ti, sl):
        be_a = 2 * p
        t8 = _T8[ti]
        src_a = tab_ref.at[idx_v.at[pl.ds(be_a * _TP + t8, 8)]]
        src_b = tab_ref.at[idx_v.at[pl.ds((be_a + 1) * _TP + t8, 8)]]
        dst_a = bufs[sl].at[pl.ds(0, 8)]
        dst_b = bufs[sl].at[pl.ds(8, 8)]
        return (src_a, dst_a), (src_b, dst_b)

    def fire_gathers(p, ti, sl):
        for src, dst in halves(p, ti, sl):
            pltpu.async_copy(src, dst, gsems[sl])

    def wait_gathers(p, ti, sl):
        for src, dst in halves(p, ti, sl):
            pltpu.make_async_copy(src, dst, gsems[sl]).wait()

    def store_pairs(p, ti, sl):
        t8, nr = _T8[ti], _SR[ti]
        be_a = 2 * p
        yield (bufs[sl].at[pl.ds(0, nr)],
               out_ref.at[b_base + be_a, pl.ds(t8, nr)])
        yield (bufs[sl].at[pl.ds(8, nr)],
               out_ref.at[b_base + be_a + 1, pl.ds(t8, nr)])

    def fire_stores(p, ti, sl):
        for src, dst in store_pairs(p, ti, sl):
            pltpu.async_copy(src, dst, ssems[sl])

    def wait_stores(p, ti, sl):
        for src, dst in store_pairs(p, ti, sl):
            pltpu.make_async_copy(src, dst, ssems[sl]).wait()

    def add_pos(ti, sl):
        t8, nr = _T8[ti], _SR[ti]

        cur = tuple(
            pos_v[pl.ds((t8 + r) * _D, _LANES)] for r in range(nr))

        def dv_body(dv, cur):
            off = dv * _LANES
            nxt = []
            for r in range(nr):
                plsc.addupdate(bufs[sl].at[r, pl.ds(off, _LANES)], cur[r])
                plsc.addupdate(bufs[sl].at[8 + r, pl.ds(off, _LANES)],
                               cur[r])
                nxt.append(pos_v[pl.ds(
                    (t8 + r) * _D + off + _LANES, _LANES)])
            return tuple(nxt)

        cur = lax.fori_loop(0, _DV - 1, dv_body, cur)
        off = (_DV - 1) * _LANES
        for r in range(nr):
            plsc.addupdate(bufs[sl].at[r, pl.ds(off, _LANES)], cur[r])
            plsc.addupdate(bufs[sl].at[8 + r, pl.ds(off, _LANES)], cur[r])

    for ti in range(_NSLOT):
        fire_gathers(0, ti, ti)

    def pair_body(p, carry):
        for ti in range(_NT):
            c = p * _NT + ti
            sl = ti % _NSLOT
            tir = (ti - 2) % _NT
            pr = p - (1 if ti < 2 else 0)
            slr = tir % _NSLOT
            tif = (ti + 3) % _NT
            pf = p + (1 if ti >= _NT - 3 else 0)

            @pl.when(c >= 2)
            def _retire():
                wait_stores(pr, tir, slr)

            @pl.when(jnp.logical_and(c >= 2, c + 3 <= _NCHUNK - 1))
            def _refill():
                fire_gathers(pf, tif, slr)

            wait_gathers(p, ti, sl)
            add_pos(ti, sl)
            fire_stores(p, ti, sl)

        return carry

    lax.fori_loop(0, _NPAIR, pair_body, 0)

    wait_stores(_NPAIR - 1, _NT - 2, (_NT - 2) % _NSLOT)
    wait_stores(_NPAIR - 1, _NT - 1, (_NT - 1) % _NSLOT)


def kernel(tokens, token_embd, position_embd):
    tokens_pad = jnp.pad(tokens.astype(jnp.int32), ((0, 0), (0, _TP - _T)))
    return _embed_sc(tokens_pad.reshape(-1), position_embd.reshape(-1),
                     token_embd)


# speedup vs baseline: 1.1174x; 1.0583x over previous
import functools

import jax
import jax.numpy as jnp
from jax import lax
from jax.experimental import pallas as pl
from jax.experimental.pallas import tpu as pltpu
from jax.experimental.pallas import tpu_sc as plsc

_V = 49408
_D = 768
_T = 77
_B = 1024
_NW = 32
_ROWS = _B * _T
_RPW = _ROWS // _NW
_CHUNK = 32
_NCH = _RPW // _CHUNK
_NSLOT = 4
_NBLK = 76 // _NSLOT
_LANES = 16
_DV = _D // _LANES

_mesh = plsc.VectorSubcoreMesh(core_axis_name="c", subcore_axis_name="s")


@functools.partial(
    pl.kernel,
    out_type=jax.ShapeDtypeStruct((_ROWS, _D), jnp.float32),
    mesh=_mesh,
    scratch_types=(
        [pltpu.VMEM((_RPW,), jnp.int32),
         pltpu.VMEM((4 * _D,), jnp.float32)]
        + [pltpu.VMEM((_CHUNK, _D), jnp.float32) for _ in range(_NSLOT)]
        + [pltpu.VMEM((_CHUNK,), jnp.int32) for _ in range(_NSLOT)]
        + [pltpu.SemaphoreType.DMA for _ in range(2 * _NSLOT)]
    ),
)
def _embed_sc(tok_ref, pos_ref, tab_ref, out_ref, idx_v, pos_v, *rest):
    bufs = rest[:_NSLOT]
    oidx = rest[_NSLOT:2 * _NSLOT]
    gsems = rest[2 * _NSLOT:3 * _NSLOT]
    ssems = rest[3 * _NSLOT:4 * _NSLOT]

    wid = lax.axis_index("s") * 2 + lax.axis_index("c")
    base = wid * _RPW
    t0 = base // _B

    pltpu.sync_copy(tok_ref.at[pl.ds(base, _RPW)], idx_v)
    pltpu.sync_copy(pos_ref.at[pl.ds(t0 * _D, 4 * _D)], pos_v)

    def fire_gather(k, sl):
        pltpu.async_copy(
            tab_ref.at[idx_v.at[pl.ds(k * _CHUNK, _CHUNK)]], bufs[sl],
            gsems[sl])

    def wait_gather(k, sl):
        pltpu.make_async_copy(
            tab_ref.at[idx_v.at[pl.ds(k * _CHUNK, _CHUNK)]], bufs[sl],
            gsems[sl]).wait()

    def wait_store(sl):
        pltpu.make_async_copy(bufs[sl], out_ref.at[oidx[sl]],
                              ssems[sl]).wait()

    def process(k, sl):
        g = base + k * _CHUNK
        t = g // _B
        b0 = g % _B
        ti = t - t0
        row0 = b0 * _T + t
        i16 = lax.iota(jnp.int32, _LANES) * _T
        oidx[sl][pl.ds(0, _LANES)] = i16 + row0
        oidx[sl][pl.ds(_LANES, _LANES)] = i16 + (row0 + _LANES * _T)

        wait_gather(k, sl)

        def dv_body(dv, carry):
            off = dv * _LANES
            pv = pos_v[pl.ds(ti * _D + off, _LANES)]
            for i in range(_CHUNK):
                plsc.addupdate(bufs[sl].at[i, pl.ds(off, _LANES)], pv)
            return carry
        lax.fori_loop(0, _DV, dv_body, 0)

        pltpu.async_copy(bufs[sl], out_ref.at[oidx[sl]], ssems[sl])

    for sl in range(_NSLOT):
        fire_gather(sl, sl)

    def block(o, carry):
        for s in range(_NSLOT):
            k = o * _NSLOT + s
            process(k, s)
            s3 = (s - 1) % _NSLOT

            @pl.when(jnp.logical_and(k >= 1, k + 3 <= _NCH - 1))
            def _retire_refill():
                wait_store(s3)
                fire_gather(k + 3, s3)

        return carry

    lax.fori_loop(0, _NBLK, block, 0)

    process(_NCH - 1, 0)

    for sl in (1, 2, 3, 0):
        wait_store(sl)


def kernel(tokens, token_embd, position_embd):
    tokens_t = tokens.astype(jnp.int32).T.reshape(-1)
    pos_flat = jnp.pad(position_embd, ((0, 3), (0, 0))).reshape(-1)
    out = _embed_sc(tokens_t, pos_flat, token_embd)
    return out.reshape(_B, _T, _D)
